# baseline (speedup 1.0000x reference)
import jax
import jax.numpy as jnp
from jax import lax
from jax.experimental import pallas as pl
from jax.experimental.pallas import tpu as pltpu
from jax.experimental.pallas import tpu_sc as plsc

_OLD_VOCAB = 1000000
_NEW_VOCAB = 1000
_EMBED_DIM = 64

_NUM_WORKERS = 32
_HALF = 256
_SUB = 128
_LANES = 16


def _body(ids_hbm, old_hbm, new_hbm, out_hbm,
          idbuf_v, idxbuf_v, buf_v, bufT0_v, bufT1_v, nidx_v, nrows_v,
          isem, gsem0, gsem1, wsem0, wsem1, fsem):
    n_seq = ids_hbm.shape[0]
    batch = ids_hbm.shape[1]
    per_w = batch // _NUM_WORKERS
    wid = lax.axis_index("s") * 2 + lax.axis_index("c")
    wb = wid * per_w

    iota16 = lax.iota(jnp.int32, _LANES)
    bufT = [bufT0_v, bufT1_v]
    gsem = [gsem0, gsem1]
    wsem = [wsem0, wsem1]

    def ids_slice(s):
        return ids_hbm.at[s, pl.ds(wb, per_w)]

    def idbuf(p):
        return idbuf_v.at[pl.ds(p * per_w, per_w)]

    def fire_ids(s, p):
        pltpu.async_copy(ids_slice(s), idbuf(p), isem)

    def wait_ids(s, p):
        pltpu.make_async_copy(ids_slice(s), idbuf(p), isem).wait()

    def clip(p):
        def clip_body(i, _):
            v = idbuf_v[pl.ds(p * per_w + i * _LANES, _LANES)]
            idxbuf_v[pl.ds(p * per_w + i * _LANES, _LANES)] = (
                jnp.minimum(v, _OLD_VOCAB - 1))
            return 0
        lax.fori_loop(0, per_w // _LANES, clip_body, 0)

    def gather_pairs(p, h):
        out = []
        for j in range(_HALF // _SUB):
            off = p * per_w + h * _HALF + j * _SUB
            out.append((old_hbm.at[idxbuf_v.at[pl.ds(off, _SUB)]],
                        buf_v.at[pl.ds(h * _HALF + j * _SUB, _SUB)]))
        return out

    def fire_gathers(p, h):
        for src, dst in gather_pairs(p, h):
            pltpu.async_copy(src, dst, gsem[h])

    def wait_gathers(p, h):
        for src, dst in gather_pairs(p, h):
            pltpu.make_async_copy(src, dst, gsem[h]).wait()

    def fixup(p, h):
        def fix_body(i, _):
            v = idbuf_v[pl.ds(p * per_w + h * _HALF + i * _LANES, _LANES)]
            m = v >= _OLD_VOCAB
            cnt = plsc.all_reduce_population_count(m)

            @pl.when(cnt[0] > 0)
            def _():
                nid = jnp.clip(v - _OLD_VOCAB, 0, _NEW_VOCAB - 1)
                nidx_v[...] = nid
                pltpu.async_copy(new_hbm.at[nidx_v], nrows_v, fsem).wait()
                rowpos = iota16 + (h * _HALF + i * _LANES)
                for d in range(_EMBED_DIM):
                    dvec = jnp.full((_LANES,), d, jnp.int32)
                    vals = plsc.load_gather(nrows_v, [iota16, dvec], mask=m)
                    plsc.store_scatter(buf_v, [rowpos, dvec], vals, mask=m)
            return 0
        lax.fori_loop(0, _HALF // _LANES, fix_body, 0)

    jvecs = [[iota16 + (h * _HALF + j * _LANES) for j in range(_HALF // _LANES)]
             for h in range(2)]

    def transpose(h):
        dst = bufT[h]

        def t_body(i, _):
            for q in range(4):
                d = i * 4 + q
                dvec = jnp.full((_LANES,), d, jnp.int32)
                for j in range(_HALF // _LANES):
                    vals = plsc.load_gather(buf_v, [jvecs[h][j], dvec])
                    dst[d, pl.ds(j * _LANES, _LANES)] = vals
            return 0
        lax.fori_loop(0, _EMBED_DIM // 4, t_body, 0)

    def out_slice(s, h):
        return out_hbm.at[s, :, pl.ds(wb + h * _HALF, _HALF)]

    def fire_write(s, h):
        pltpu.async_copy(bufT[h], out_slice(s, h), wsem[h])

    def wait_write(s, h):
        pltpu.make_async_copy(bufT[h], out_slice(s, h), wsem[h]).wait()

    fire_ids(0, 0)
    wait_ids(0, 0)
    clip(0)
    fire_gathers(0, 0)
    fire_gathers(0, 1)
    fire_ids(1, 1)

    def pair(sp, _):
        for sb in range(2):
            s = sp * 2 + sb
            p = sb
            np_ = 1 - sb

            @pl.when(s + 1 < n_seq)
            def _():
                wait_ids(s + 1, np_)
                clip(np_)

            for h in range(2):
                wait_gathers(p, h)
                fixup(p, h)

                @pl.when(s >= 1)
                def _():
                    wait_write(s - 1, h)
                transpose(h)
                fire_write(s, h)

                @pl.when(s + 1 < n_seq)
                def _():
                    fire_gathers(np_, h)

            @pl.when(s + 2 < n_seq)
            def _():
                fire_ids(s + 2, p)
        return 0

    lax.fori_loop(0, n_seq // 2, pair, 0)
    wait_write(n_seq - 1, 0)
    wait_write(n_seq - 1, 1)


def kernel(input_ids, old_table, new_table):
    batch, seq = input_ids.shape
    ids_t = input_ids.T

    mesh = plsc.VectorSubcoreMesh(core_axis_name="c", subcore_axis_name="s")
    run = pl.kernel(
        _body,
        out_type=jax.ShapeDtypeStruct((seq, _EMBED_DIM, batch), jnp.float32),
        mesh=mesh,
        compiler_params=pltpu.CompilerParams(
            needs_layout_passes=False, use_tc_tiling_on_sc=False),
        scratch_types=[
            pltpu.VMEM((2 * 512,), jnp.int32),
            pltpu.VMEM((2 * 512,), jnp.int32),
            pltpu.VMEM((512, _EMBED_DIM), jnp.float32),
            pltpu.VMEM((_EMBED_DIM, _HALF), jnp.float32),
            pltpu.VMEM((_EMBED_DIM, _HALF), jnp.float32),
            pltpu.VMEM((_LANES,), jnp.int32),
            pltpu.VMEM((_LANES, _EMBED_DIM), jnp.float32),
            pltpu.SemaphoreType.DMA,
            pltpu.SemaphoreType.DMA,
            pltpu.SemaphoreType.DMA,
            pltpu.SemaphoreType.DMA,
            pltpu.SemaphoreType.DMA,
            pltpu.SemaphoreType.DMA,
        ],
    )
    out_t = run(ids_t, old_table, new_table)
    return jnp.transpose(out_t, (2, 0, 1))


# speedup vs baseline: 1.0682x; 1.0682x over previous
import functools

import jax
import jax.numpy as jnp
from jax import lax
from jax.experimental import pallas as pl
from jax.experimental.pallas import tpu as pltpu
from jax.experimental.pallas import tpu_sc as plsc

_OLD_VOCAB = 1000000
_NEW_VOCAB = 1000
_EMBED_DIM = 64

_NUM_WORKERS = 32
_CHUNK = 256
_SUB = 128
_LANES = 16
_NBUF = 2

_TC_BLOCK = 512


def _sc_body(ids_hbm, old_hbm, new_hbm, out_hbm,
             newtab_v, raw_v, idx_v, rows_v, gsems, wsems):
    n_ids = ids_hbm.shape[0]
    per_w = n_ids // _NUM_WORKERS
    n_chunks = per_w // _CHUNK
    n_steps = n_chunks // _NBUF
    wid = lax.axis_index("s") * 2 + lax.axis_index("c")
    base_w = wid * per_w

    pltpu.sync_copy(new_hbm, newtab_v)

    iota16 = lax.iota(jnp.int32, _LANES)

    def raw(b):
        return raw_v.at[pl.ds(b * _CHUNK, _CHUNK)]

    def idx(b):
        return idx_v.at[pl.ds(b * _CHUNK, _CHUNK)]

    def rows(b):
        return rows_v.at[pl.ds(b * _CHUNK, _CHUNK)]

    def load_and_clip(c, b):
        base = base_w + c * _CHUNK
        pltpu.sync_copy(ids_hbm.at[pl.ds(base, _CHUNK)], raw(b))

        def clip_body(i, _):
            v = raw_v[pl.ds(b * _CHUNK + i * _LANES, _LANES)]
            idx_v[pl.ds(b * _CHUNK + i * _LANES, _LANES)] = (
                jnp.minimum(v, _OLD_VOCAB - 1))
            return 0
        lax.fori_loop(0, _CHUNK // _LANES, clip_body, 0)

    def fire_gathers(b):
        for j in range(_CHUNK // _SUB):
            pltpu.async_copy(
                old_hbm.at[idx(b).at[pl.ds(j * _SUB, _SUB)]],
                rows(b).at[pl.ds(j * _SUB, _SUB)], gsems[b])

    def wait_gathers(b):
        for j in range(_CHUNK // _SUB):
            pltpu.make_async_copy(
                old_hbm.at[idx(b).at[pl.ds(j * _SUB, _SUB)]],
                rows(b).at[pl.ds(j * _SUB, _SUB)], gsems[b]).wait()

    def fixup(b):
        def fix_body(i, _):
            v = raw_v[pl.ds(b * _CHUNK + i * _LANES, _LANES)]
            m = v >= _OLD_VOCAB
            cnt = plsc.all_reduce_population_count(m)

            @pl.when(cnt[0] > 0)
            def _():
                nid = jnp.maximum(v - _OLD_VOCAB, 0)
                rowpos = iota16 + (b * _CHUNK + i * _LANES)

                def d_body(d, _):
                    dvec = jnp.full((_LANES,), d, jnp.int32)
                    vals = plsc.load_gather(newtab_v, [nid, dvec], mask=m)
                    plsc.store_scatter(rows_v, [rowpos, dvec], vals, mask=m)
                    return 0
                lax.fori_loop(0, _EMBED_DIM, d_body, 0)
            return 0
        lax.fori_loop(0, _CHUNK // _LANES, fix_body, 0)

    for b in range(_NBUF):
        load_and_clip(b, b)
        fire_gathers(b)

    def step(g, _):
        for b in range(_NBUF):
            c = g * _NBUF + b
            wait_gathers(b)
            fixup(b)
            wdesc = pltpu.make_async_copy(
                rows(b), out_hbm.at[pl.ds(base_w + c * _CHUNK, _CHUNK)],
                wsems[b])
            wdesc.start()

            @pl.when(g < n_steps - 1)
            def _():
                load_and_clip(c + _NBUF, b)
            wdesc.wait()

            @pl.when(g < n_steps - 1)
            def _():
                fire_gathers(b)
        return 0

    lax.fori_loop(0, n_steps, step, 0)


def _sc_gather(flat_ids, old_table, new_table):
    n_ids = flat_ids.shape[0]
    mesh = plsc.VectorSubcoreMesh(core_axis_name="c", subcore_axis_name="s")
    run = pl.kernel(
        _sc_body,
        out_type=jax.ShapeDtypeStruct((n_ids, _EMBED_DIM), jnp.float32),
        mesh=mesh,
        compiler_params=pltpu.CompilerParams(
            needs_layout_passes=False, use_tc_tiling_on_sc=False),
        scratch_types=[
            pltpu.VMEM((_NEW_VOCAB, _EMBED_DIM), jnp.float32),
            pltpu.VMEM((_NBUF * _CHUNK,), jnp.int32),
            pltpu.VMEM((_NBUF * _CHUNK,), jnp.int32),
            pltpu.VMEM((_NBUF * _CHUNK, _EMBED_DIM), jnp.float32),
            [pltpu.SemaphoreType.DMA] * _NBUF,
            [pltpu.SemaphoreType.DMA] * _NBUF,
        ],
    )
    return run(flat_ids, old_table, new_table)


def _tc_transpose_body(in_ref, out_ref):
    out_ref[...] = jnp.transpose(in_ref[...], (1, 0))[None]


def _tc_transpose(flat_emb, seq, batch):
    grid = (seq, batch // _TC_BLOCK)
    return pl.pallas_call(
        _tc_transpose_body,
        grid=grid,
        in_specs=[pl.BlockSpec(
            (_TC_BLOCK, _EMBED_DIM),
            lambda s, bb: (s * (batch // _TC_BLOCK) + bb, 0))],
        out_specs=pl.BlockSpec(
            (1, _EMBED_DIM, _TC_BLOCK), lambda s, bb: (s, 0, bb)),
        out_shape=jax.ShapeDtypeStruct((seq, _EMBED_DIM, batch), jnp.float32),
    )(flat_emb)


def kernel(input_ids, old_table, new_table):
    batch, seq = input_ids.shape
    flat_ids = input_ids.T.reshape(seq * batch)
    flat_emb = _sc_gather(flat_ids, old_table, new_table)
    out_t = _tc_transpose(flat_emb, seq, batch)
    return jnp.transpose(out_t, (2, 0, 1))


# speedup vs baseline: 2.0925x; 1.9588x over previous
import jax
import jax.numpy as jnp
from jax import lax
from jax.experimental import pallas as pl
from jax.experimental.pallas import tpu as pltpu
from jax.experimental.pallas import tpu_sc as plsc

_OLD_VOCAB = 1000000
_NEW_VOCAB = 1000
_EMBED_DIM = 64

_NUM_WORKERS = 32
_ROWS = 2
_LANES = 16
_NBUF = 2

_GROUP_OFFS = tuple(range(0, 192, 16)) + (184,)
_GATHER_SPLITS = ((0, 128), (128, 72))


def _sc_body(ids_hbm, old_hbm, new_hbm, out_hbm,
             newtab_v, raw_v, idx_v, rows_v, gsems, wsems):
    batch = ids_hbm.shape[0]
    seq = ids_hbm.shape[1]
    per_w = batch // _NUM_WORKERS
    n_chunks = per_w // _ROWS
    n_steps = n_chunks // _NBUF
    wid = lax.axis_index("s") * 2 + lax.axis_index("c")
    base_row = wid * per_w

    pltpu.sync_copy(new_hbm, newtab_v)

    iota16 = lax.iota(jnp.int32, _LANES)

    def load_and_clip(c, b):
        row0 = base_row + c * _ROWS
        pltpu.sync_copy(ids_hbm.at[pl.ds(row0, _ROWS), :],
                        raw_v.at[pl.ds(b * _ROWS, _ROWS), :])
        for rr in range(_ROWS):
            for off in _GROUP_OFFS:
                v = raw_v[b * _ROWS + rr, pl.ds(off, _LANES)]
                idx_v[b * _ROWS + rr, pl.ds(off, _LANES)] = (
                    jnp.minimum(v, _OLD_VOCAB - 1))

    def gather_pairs(b):
        out = []
        for rr in range(_ROWS):
            for off, ln in _GATHER_SPLITS:
                out.append((
                    old_hbm.at[idx_v.at[b * _ROWS + rr, pl.ds(off, ln)]],
                    rows_v.at[b * _ROWS + rr, pl.ds(off, ln)]))
        return out

    def fire_gathers(b):
        for src, dst in gather_pairs(b):
            pltpu.async_copy(src, dst, gsems[b])

    def wait_gathers(b):
        for src, dst in gather_pairs(b):
            pltpu.make_async_copy(src, dst, gsems[b]).wait()

    def fixup(b):
        for rr in range(_ROWS):
            for off in _GROUP_OFFS:
                v = raw_v[b * _ROWS + rr, pl.ds(off, _LANES)]
                m = v >= _OLD_VOCAB
                cnt = plsc.all_reduce_population_count(m)

                @pl.when(cnt[0] > 0)
                def _(rr=rr, off=off, v=v, m=m):
                    nid = jnp.maximum(v - _OLD_VOCAB, 0)
                    rvec = jnp.full((_LANES,), b * _ROWS + rr, jnp.int32)
                    pvec = iota16 + off

                    def d_body(d, _):
                        dvec = jnp.full((_LANES,), d, jnp.int32)
                        vals = plsc.load_gather(newtab_v, [nid, dvec], mask=m)
                        plsc.store_scatter(rows_v, [rvec, pvec, dvec], vals,
                                           mask=m)
                        return 0
                    lax.fori_loop(0, _EMBED_DIM, d_body, 0)

    for b in range(_NBUF):
        load_and_clip(b, b)
        fire_gathers(b)

    def step(g, _):
        for b in range(_NBUF):
            c = g * _NBUF + b
            wait_gathers(b)
            fixup(b)
            row0 = base_row + c * _ROWS
            wdesc = pltpu.make_async_copy(
                rows_v.at[pl.ds(b * _ROWS, _ROWS)],
                out_hbm.at[pl.ds(row0, _ROWS)], wsems[b])
            wdesc.start()

            @pl.when(g < n_steps - 1)
            def _():
                load_and_clip(c + _NBUF, b)
            wdesc.wait()

            @pl.when(g < n_steps - 1)
            def _():
                fire_gathers(b)
        return 0

    lax.fori_loop(0, n_steps, step, 0)


def kernel(input_ids, old_table, new_table):
    batch, seq = input_ids.shape
    mesh = plsc.VectorSubcoreMesh(core_axis_name="c", subcore_axis_name="s")
    run = pl.kernel(
        _sc_body,
        out_type=jax.ShapeDtypeStruct((batch, seq, _EMBED_DIM), jnp.float32),
        mesh=mesh,
        compiler_params=pltpu.CompilerParams(
            needs_layout_passes=False, use_tc_tiling_on_sc=False),
        scratch_types=[
            pltpu.VMEM((_NEW_VOCAB, _EMBED_DIM), jnp.float32),
            pltpu.VMEM((_NBUF * _ROWS, seq), jnp.int32),
            pltpu.VMEM((_NBUF * _ROWS, seq), jnp.int32),
            pltpu.VMEM((_NBUF * _ROWS, seq, _EMBED_DIM), jnp.float32),
            [pltpu.SemaphoreType.DMA] * _NBUF,
            [pltpu.SemaphoreType.DMA] * _NBUF,
        ],
    )
    return run(input_ids, old_table, new_table)


# speedup vs baseline: 2.1426x; 1.0239x over previous
import jax
import jax.numpy as jnp
from jax import lax
from jax.experimental import pallas as pl
from jax.experimental.pallas import tpu as pltpu
from jax.experimental.pallas import tpu_sc as plsc

_OLD_VOCAB = 1000000
_NEW_VOCAB = 1000
_EMBED_DIM = 64

_NUM_WORKERS = 32
_CHUNK = 256
_SUB = 128
_LANES = 16
_NBUF = 2


def _body(ids_hbm, old_hbm, new_hbm, out_hbm,
          newtab_v, raw_v, idx_v, rows_v, gsems, wsems):
    n_ids = ids_hbm.shape[0]
    per_w = n_ids // _NUM_WORKERS
    n_chunks = per_w // _CHUNK
    n_steps = n_chunks // _NBUF
    wid = lax.axis_index("s") * 2 + lax.axis_index("c")
    base_w = wid * per_w

    pltpu.sync_copy(new_hbm, newtab_v)

    iota16 = lax.iota(jnp.int32, _LANES)

    def raw(b):
        return raw_v.at[pl.ds(b * _CHUNK, _CHUNK)]

    def idx(b):
        return idx_v.at[pl.ds(b * _CHUNK, _CHUNK)]

    def rows(b):
        return rows_v.at[pl.ds(b * _CHUNK, _CHUNK)]

    def load_and_clip(c, b):
        base = base_w + c * _CHUNK
        pltpu.sync_copy(ids_hbm.at[pl.ds(base, _CHUNK)], raw(b))

        def clip_body(i, _):
            v = raw_v[pl.ds(b * _CHUNK + i * _LANES, _LANES)]
            idx_v[pl.ds(b * _CHUNK + i * _LANES, _LANES)] = (
                jnp.minimum(v, _OLD_VOCAB - 1))
            return 0
        lax.fori_loop(0, _CHUNK // _LANES, clip_body, 0)

    def fire_gathers(b):
        for j in range(_CHUNK // _SUB):
            pltpu.async_copy(
                old_hbm.at[idx(b).at[pl.ds(j * _SUB, _SUB)]],
                rows(b).at[pl.ds(j * _SUB, _SUB)], gsems[b])

    def wait_gathers(b):
        for j in range(_CHUNK // _SUB):
            pltpu.make_async_copy(
                old_hbm.at[idx(b).at[pl.ds(j * _SUB, _SUB)]],
                rows(b).at[pl.ds(j * _SUB, _SUB)], gsems[b]).wait()

    def fixup(b):
        def fix_body(i, _):
            v = raw_v[pl.ds(b * _CHUNK + i * _LANES, _LANES)]
            m = v >= _OLD_VOCAB
            cnt = plsc.all_reduce_population_count(m)

            @pl.when(cnt[0] > 0)
            def _():
                nid = jnp.maximum(v - _OLD_VOCAB, 0)
                rowpos = iota16 + (b * _CHUNK + i * _LANES)

                def d_body(d, _):
                    dvec = jnp.full((_LANES,), d, jnp.int32)
                    vals = plsc.load_gather(newtab_v, [nid, dvec], mask=m)
                    plsc.store_scatter(rows_v, [rowpos, dvec], vals, mask=m)
                    return 0
                lax.fori_loop(0, _EMBED_DIM, d_body, 0)
            return 0
        lax.fori_loop(0, _CHUNK // _LANES, fix_body, 0)

    for b in range(_NBUF):
        load_and_clip(b, b)
        fire_gathers(b)

    def step(g, _):
        for b in range(_NBUF):
            c = g * _NBUF + b
            wait_gathers(b)
            fixup(b)
            wdesc = pltpu.make_async_copy(
                rows(b), out_hbm.at[pl.ds(base_w + c * _CHUNK, _CHUNK)],
                wsems[b])
            wdesc.start()

            @pl.when(g < n_steps - 1)
            def _():
                load_and_clip(c + _NBUF, b)
            wdesc.wait()

            @pl.when(g < n_steps - 1)
            def _():
                fire_gathers(b)
        return 0

    lax.fori_loop(0, n_steps, step, 0)


def kernel(input_ids, old_table, new_table):
    batch, seq = input_ids.shape
    n_ids = batch * seq
    flat_ids = input_ids.reshape(n_ids)

    mesh = plsc.VectorSubcoreMesh(core_axis_name="c", subcore_axis_name="s")
    run = pl.kernel(
        _body,
        out_type=jax.ShapeDtypeStruct((n_ids, _EMBED_DIM), jnp.float32),
        mesh=mesh,
        compiler_params=pltpu.CompilerParams(
            needs_layout_passes=False, use_tc_tiling_on_sc=False),
        scratch_types=[
            pltpu.VMEM((_NEW_VOCAB, _EMBED_DIM), jnp.float32),
            pltpu.VMEM((_NBUF * _CHUNK,), jnp.int32),
            pltpu.VMEM((_NBUF * _CHUNK,), jnp.int32),
            pltpu.VMEM((_NBUF * _CHUNK, _EMBED_DIM), jnp.float32),
            [pltpu.SemaphoreType.DMA] * _NBUF,
            [pltpu.SemaphoreType.DMA] * _NBUF,
        ],
    )
    out = run(flat_ids, old_table, new_table)
    return out.reshape(batch, seq, _EMBED_DIM)
